# Initial kernel scaffold; baseline (speedup 1.0000x reference)
import jax
import jax.numpy as jnp
from jax.experimental import pallas as pl


def kernel(boxes, feat0, feat1, feat2, feat3, feat4, image_shape):
    raise NotImplementedError("write your pallas kernel here")


# speedup vs baseline: 16.6136x; 16.6136x over previous
import functools

import jax
import jax.numpy as jnp
import numpy as np
from jax import lax
from jax.experimental import pallas as pl
from jax.experimental.pallas import tpu as pltpu
from jax.experimental.pallas import tpu_sc as plsc

CROP = 7
PIX = CROP * CROP
C = 256
SIZES = (256, 128, 64, 32, 16)
ROWS_PER_BATCH = sum(s * s for s in SIZES)
NBOX_PAD = 2048
M_PAD = NBOX_PAD * PIX
NC, NS, L = 2, 16, 16
NW = NC * NS
PX_PER_W = M_PAD // NW
CHUNK = 32
NCHUNK = PX_PER_W // CHUNK


def _prep_body(b_ref, bidx_ref, c2_ref, idx_ref, w_ref):
    x1 = b_ref[0]
    y1 = b_ref[1]
    x2 = b_ref[2]
    y2 = b_ref[3]
    h = y2 - y1
    w = x2 - x1
    c2 = c2_ref[0, 0]
    lvlf = jnp.log(jnp.sqrt(h * w) / c2) / np.float32(np.log(2.0))
    lvl = jnp.minimum(4, jnp.maximum(0, jnp.round(lvlf).astype(jnp.int32)))
    s_i = jnp.int32(256) >> lvl
    hm1 = s_i.astype(jnp.float32) - 1.0
    off = ((lvl >= 1) * 65536 + (lvl >= 2) * 16384
           + (lvl >= 3) * 4096 + (lvl >= 4) * 1024)
    rowbase = bidx_ref[...] * ROWS_PER_BATCH + off
    y1b, x1b, y2b, x2b = x1, y1, x2, y2
    sy = (y2b - y1b) * hm1 / np.float32(CROP - 1)
    sx = (x2b - x1b) * hm1 / np.float32(CROP - 1)
    base_y = y1b * hm1
    base_x = x1b * hm1
    zero = jnp.float32(0.0)
    for i in range(CROP):
        ys = base_y + np.float32(i) * sy
        y0f = jnp.floor(ys)
        wy = ys - y0f
        y0 = jnp.clip(y0f, zero, hm1).astype(jnp.int32)
        y1c = jnp.clip(y0f + 1.0, zero, hm1).astype(jnp.int32)
        ry0 = rowbase + y0 * s_i
        ry1 = rowbase + y1c * s_i
        for j in range(CROP):
            xs = base_x + np.float32(j) * sx
            x0f = jnp.floor(xs)
            wx = xs - x0f
            x0 = jnp.clip(x0f, zero, hm1).astype(jnp.int32)
            x1c = jnp.clip(x0f + 1.0, zero, hm1).astype(jnp.int32)
            k = i * CROP + j
            idx_ref[k, 0] = ry0 + x0
            idx_ref[k, 1] = ry0 + x1c
            idx_ref[k, 2] = ry1 + x0
            idx_ref[k, 3] = ry1 + x1c
            w_ref[k, 0] = (1.0 - wy) * (1.0 - wx)
            w_ref[k, 1] = (1.0 - wy) * wx
            w_ref[k, 2] = wy * (1.0 - wx)
            w_ref[k, 3] = wy * wx


def _sc_body(table, idxf, wf, out, idx_v, w_v, rows_v, out_v, sem):
    wid = lax.axis_index("s") * NC + lax.axis_index("c")
    base_px = wid * PX_PER_W

    def chunk_body(ci, carry):
        pbase = base_px + ci * CHUNK
        pltpu.sync_copy(idxf.at[pl.ds(pbase * 4, CHUNK * 4)], idx_v)
        pltpu.sync_copy(wf.at[pl.ds(pbase * 4, CHUNK * 4)],
                        w_v.at[pl.ds(0, CHUNK * 4)])
        pltpu.async_copy(table.at[idx_v], rows_v, sem).wait()

        def px_body(p, c2):
            wvec = w_v[pl.ds(4 * p, L)]
            w0 = wvec[0]
            w1 = wvec[1]
            w2 = wvec[2]
            w3 = wvec[3]
            for q in range(C // L):
                col = pl.ds(q * L, L)
                acc = (w0 * rows_v[4 * p, col]
                       + w1 * rows_v[4 * p + 1, col]
                       + w2 * rows_v[4 * p + 2, col]
                       + w3 * rows_v[4 * p + 3, col])
                out_v[p, col] = acc
            return c2

        lax.fori_loop(0, CHUNK, px_body, 0)
        pltpu.sync_copy(out_v, out.at[pl.ds(pbase, CHUNK)])
        return carry

    lax.fori_loop(0, NCHUNK, chunk_body, 0)


@jax.jit
def kernel(boxes, feat0, feat1, feat2, feat3, feat4, image_shape):
    B, N = boxes.shape[0], boxes.shape[1]
    feats = (feat0, feat1, feat2, feat3, feat4)
    table = jnp.concatenate(
        [f.reshape(B, -1, C) for f in feats], axis=1).reshape(-1, C)

    fb = boxes.reshape(B * N, 4)
    fb = jnp.pad(fb, ((0, NBOX_PAD - B * N), (0, 0)))
    b_in = fb.T.reshape(4, 16, 128)
    bidx = jnp.pad(jnp.repeat(jnp.arange(B, dtype=jnp.int32), N),
                   (0, NBOX_PAD - B * N)).reshape(16, 128)
    area = (image_shape[0] * image_shape[1]).astype(jnp.float32)
    c2 = (np.float32(56.0) / jnp.sqrt(area)).reshape(1, 1)

    idx, wts = pl.pallas_call(
        _prep_body,
        in_specs=[
            pl.BlockSpec(memory_space=pltpu.VMEM),
            pl.BlockSpec(memory_space=pltpu.VMEM),
            pl.BlockSpec(memory_space=pltpu.SMEM),
        ],
        out_specs=[
            pl.BlockSpec(memory_space=pltpu.VMEM),
            pl.BlockSpec(memory_space=pltpu.VMEM),
        ],
        out_shape=[
            jax.ShapeDtypeStruct((PIX, 4, 16, 128), jnp.int32),
            jax.ShapeDtypeStruct((PIX, 4, 16, 128), jnp.float32),
        ],
    )(b_in, bidx, c2)

    idx_flat = idx.reshape(PIX, 4, NBOX_PAD).transpose(2, 0, 1).reshape(-1)
    w_flat = wts.reshape(PIX, 4, NBOX_PAD).transpose(2, 0, 1).reshape(-1)

    mesh = plsc.VectorSubcoreMesh(core_axis_name="c", subcore_axis_name="s")
    sc_call = functools.partial(
        pl.kernel,
        out_type=jax.ShapeDtypeStruct((M_PAD, C), jnp.float32),
        mesh=mesh,
        scratch_types=[
            pltpu.VMEM((CHUNK * 4,), jnp.int32),
            pltpu.VMEM((CHUNK * 4 + L,), jnp.float32),
            pltpu.VMEM((CHUNK * 4, C), jnp.float32),
            pltpu.VMEM((CHUNK, C), jnp.float32),
            pltpu.SemaphoreType.DMA,
        ],
    )(_sc_body)
    out = sc_call(table, idx_flat, w_flat)
    return out.reshape(NBOX_PAD, PIX, C)[:B * N].reshape(
        B, N, CROP, CROP, C)
